# baseline (speedup 1.0000x reference)
import jax
import jax.numpy as jnp
from jax.experimental import pallas as pl

_BLK = 64


def _pe_kernel(me_ref, mo_ref, dlo_ref, dhi_ref, base_ref, out_ref):
    me = me_ref[...].astype(jnp.float32)
    mo = mo_ref[...].astype(jnp.float32)
    dlo = dlo_ref[...]
    dhi = dhi_ref[...]
    base = base_ref[...]
    out_ref[...] = (base[None, :, :]
                    + me[:, :, None] * dlo[None, :, :]
                    + mo[:, :, None] * dhi[None, :, :])


def kernel(context_mapping, table):
    n, l = context_mapping.shape
    d = table.shape[1]
    base = table[0:1, :]
    diff = table[:l, :] - base

    l2 = l // 2
    d2 = 2 * d
    diff2 = diff.reshape(l2, d2)
    zeros = jnp.zeros((l2, d), jnp.float32)
    dlo = jnp.concatenate([diff2[:, :d], zeros], axis=1)
    dhi = jnp.concatenate([zeros, diff2[:, d:]], axis=1)
    base2 = jnp.concatenate([base, base], axis=1)

    m_even = context_mapping[:, 0::2]
    m_odd = context_mapping[:, 1::2]

    grid = (n // _BLK,)
    out2 = pl.pallas_call(
        _pe_kernel,
        grid=grid,
        in_specs=[
            pl.BlockSpec((_BLK, l2), lambda i: (i, 0)),
            pl.BlockSpec((_BLK, l2), lambda i: (i, 0)),
            pl.BlockSpec((l2, d2), lambda i: (0, 0)),
            pl.BlockSpec((l2, d2), lambda i: (0, 0)),
            pl.BlockSpec((1, d2), lambda i: (0, 0)),
        ],
        out_specs=pl.BlockSpec((_BLK, l2, d2), lambda i: (i, 0, 0)),
        out_shape=jax.ShapeDtypeStruct((n, l2, d2), jnp.float32),
    )(m_even, m_odd, dlo, dhi, base2)
    return out2.reshape(n, l, d)


# speedup vs baseline: 1.0105x; 1.0105x over previous
import jax
import jax.numpy as jnp
from jax.experimental import pallas as pl

_BLK = 256


def _pe_kernel(me_ref, mo_ref, dlo_ref, dhi_ref, base_ref, out_ref):
    me = me_ref[...].astype(jnp.float32)
    mo = mo_ref[...].astype(jnp.float32)
    out_ref[...] = (base_ref[...][None, :, :]
                    + me[:, :, None] * dlo_ref[...][None, :, :]
                    + mo[:, :, None] * dhi_ref[...][None, :, :])


def kernel(context_mapping, table):
    n, l = context_mapping.shape
    d = table.shape[1]
    base = table[0:1, :]
    diff = table[:l, :] - base

    l2 = l // 2
    d2 = 2 * d
    diff2 = diff.reshape(l2, d2)
    zeros = jnp.zeros((l2, d), jnp.float32)
    dlo = jnp.concatenate([diff2[:, :d], zeros], axis=1)
    dhi = jnp.concatenate([zeros, diff2[:, d:]], axis=1)
    base2 = jnp.concatenate([base, base], axis=1)

    m_even = context_mapping[:, 0::2]
    m_odd = context_mapping[:, 1::2]

    grid = (n // _BLK,)
    out2 = pl.pallas_call(
        _pe_kernel,
        grid=grid,
        in_specs=[
            pl.BlockSpec((_BLK, l2), lambda i: (i, 0)),
            pl.BlockSpec((_BLK, l2), lambda i: (i, 0)),
            pl.BlockSpec((l2, d2), lambda i: (0, 0)),
            pl.BlockSpec((l2, d2), lambda i: (0, 0)),
            pl.BlockSpec((1, d2), lambda i: (0, 0)),
        ],
        out_specs=pl.BlockSpec((_BLK, l2, d2), lambda i: (i, 0, 0)),
        out_shape=jax.ShapeDtypeStruct((n, l2, d2), jnp.float32),
    )(m_even, m_odd, dlo, dhi, base2)
    return out2.reshape(n, l, d)


# speedup vs baseline: 1.0128x; 1.0023x over previous
import jax
import jax.numpy as jnp
from jax.experimental import pallas as pl
from jax.experimental.pallas import tpu as pltpu

_BLK = 256


def _pe_kernel(me_ref, mo_ref, dlo_ref, dhi_ref, base_ref, out_ref):
    me = me_ref[...].astype(jnp.float32)
    mo = mo_ref[...].astype(jnp.float32)
    out_ref[...] = (base_ref[...][None, :, :]
                    + me[:, :, None] * dlo_ref[...][None, :, :]
                    + mo[:, :, None] * dhi_ref[...][None, :, :])


def kernel(context_mapping, table):
    n, l = context_mapping.shape
    d = table.shape[1]
    base = table[0:1, :]
    diff = table[:l, :] - base

    l2 = l // 2
    d2 = 2 * d
    diff2 = diff.reshape(l2, d2)
    zeros = jnp.zeros((l2, d), jnp.float32)
    dlo = jnp.concatenate([diff2[:, :d], zeros], axis=1)
    dhi = jnp.concatenate([zeros, diff2[:, d:]], axis=1)
    base2 = jnp.concatenate([base, base], axis=1)

    m_even = context_mapping[:, 0::2]
    m_odd = context_mapping[:, 1::2]

    grid = (n // _BLK,)
    out2 = pl.pallas_call(
        _pe_kernel,
        grid=grid,
        in_specs=[
            pl.BlockSpec((_BLK, l2), lambda i: (i, 0)),
            pl.BlockSpec((_BLK, l2), lambda i: (i, 0)),
            pl.BlockSpec((l2, d2), lambda i: (0, 0)),
            pl.BlockSpec((l2, d2), lambda i: (0, 0)),
            pl.BlockSpec((1, d2), lambda i: (0, 0)),
        ],
        out_specs=pl.BlockSpec((_BLK, l2, d2), lambda i: (i, 0, 0)),
        out_shape=jax.ShapeDtypeStruct((n, l2, d2), jnp.float32),
        compiler_params=pltpu.CompilerParams(
            dimension_semantics=("parallel",)),
    )(m_even, m_odd, dlo, dhi, base2)
    return out2.reshape(n, l, d)
